# Initial kernel scaffold; baseline (speedup 1.0000x reference)
import jax
import jax.numpy as jnp
from jax.experimental import pallas as pl


def kernel(cov, embeddings, noise_scale):
    raise NotImplementedError("write your pallas kernel here")


# speedup vs baseline: 3.3682x; 3.3682x over previous
import jax
import jax.numpy as jnp
from jax.experimental import pallas as pl

_B = 8
_N = 2048
_ROWS = 256


def _stripe_kernel(emb_ref, ns_ref, cov_ref, out_ref):
    i = pl.program_id(1)
    ev = jnp.exp(emb_ref[0]) + jnp.exp(ns_ref[0, 0])
    row = jax.lax.broadcasted_iota(jnp.int32, (_ROWS, _N), 0)
    col = jax.lax.broadcasted_iota(jnp.int32, (_ROWS, _N), 1)
    mask = col == i * _ROWS + row
    out_ref[0] = cov_ref[0] + jnp.where(mask, ev, 0.0)


def kernel(cov, embeddings, noise_scale):
    emb = embeddings[:, :, -1].reshape(_B, 1, _N)
    ns = noise_scale.reshape(1, 1)
    return pl.pallas_call(
        _stripe_kernel,
        grid=(_B, _N // _ROWS),
        in_specs=[
            pl.BlockSpec((1, 1, _N), lambda b, i: (b, 0, 0)),
            pl.BlockSpec((1, 1), lambda b, i: (0, 0)),
            pl.BlockSpec((1, _ROWS, _N), lambda b, i: (b, i, 0)),
        ],
        out_specs=pl.BlockSpec((1, _ROWS, _N), lambda b, i: (b, i, 0)),
        out_shape=jax.ShapeDtypeStruct((_B, _N, _N), jnp.float32),
    )(emb, ns, cov)


# speedup vs baseline: 3.7875x; 1.1245x over previous
import jax
import jax.numpy as jnp
from jax.experimental import pallas as pl

_B = 8
_N = 2048
_ROWS = 512


def _stripe_kernel(emb_ref, ns_ref, cov_ref, out_ref):
    i = pl.program_id(1)
    ev = jnp.exp(emb_ref[0]) + jnp.exp(ns_ref[0, 0])
    row = jax.lax.broadcasted_iota(jnp.int32, (_ROWS, _N), 0)
    col = jax.lax.broadcasted_iota(jnp.int32, (_ROWS, _N), 1)
    mask = col == i * _ROWS + row
    out_ref[0] = cov_ref[0] + jnp.where(mask, ev, 0.0)


def kernel(cov, embeddings, noise_scale):
    emb = embeddings[:, :, -1].reshape(_B, 1, _N)
    ns = noise_scale.reshape(1, 1)
    return pl.pallas_call(
        _stripe_kernel,
        grid=(_B, _N // _ROWS),
        in_specs=[
            pl.BlockSpec((1, 1, _N), lambda b, i: (b, 0, 0)),
            pl.BlockSpec((1, 1), lambda b, i: (0, 0)),
            pl.BlockSpec((1, _ROWS, _N), lambda b, i: (b, i, 0)),
        ],
        out_specs=pl.BlockSpec((1, _ROWS, _N), lambda b, i: (b, i, 0)),
        out_shape=jax.ShapeDtypeStruct((_B, _N, _N), jnp.float32),
    )(emb, ns, cov)


# speedup vs baseline: 3.8538x; 1.0175x over previous
import jax
import jax.numpy as jnp
from jax.experimental import pallas as pl

_B = 8
_N = 2048
_ROWS = 512


def _stripe_kernel(emb_ref, ns_ref, cov_ref, out_ref):
    i = pl.program_id(1)
    out_ref[0] = cov_ref[0]
    ev = jnp.exp(emb_ref[0, :, pl.ds(i * _ROWS, _ROWS)]) + jnp.exp(ns_ref[0, 0])
    row = jax.lax.broadcasted_iota(jnp.int32, (_ROWS, _ROWS), 0)
    col = jax.lax.broadcasted_iota(jnp.int32, (_ROWS, _ROWS), 1)
    sub = out_ref[0, :, pl.ds(i * _ROWS, _ROWS)]
    out_ref[0, :, pl.ds(i * _ROWS, _ROWS)] = sub + jnp.where(row == col, ev, 0.0)


def kernel(cov, embeddings, noise_scale):
    emb = embeddings[:, :, -1].reshape(_B, 1, _N)
    ns = noise_scale.reshape(1, 1)
    return pl.pallas_call(
        _stripe_kernel,
        grid=(_B, _N // _ROWS),
        in_specs=[
            pl.BlockSpec((1, 1, _N), lambda b, i: (b, 0, 0)),
            pl.BlockSpec((1, 1), lambda b, i: (0, 0)),
            pl.BlockSpec((1, _ROWS, _N), lambda b, i: (b, i, 0)),
        ],
        out_specs=pl.BlockSpec((1, _ROWS, _N), lambda b, i: (b, i, 0)),
        out_shape=jax.ShapeDtypeStruct((_B, _N, _N), jnp.float32),
    )(emb, ns, cov)


# speedup vs baseline: 3.9331x; 1.0206x over previous
import jax
import jax.numpy as jnp
from jax.experimental import pallas as pl

_B = 8
_N = 2048
_ROWS = 1024


def _stripe_kernel(emb_ref, ns_ref, cov_ref, out_ref):
    i = pl.program_id(1)
    out_ref[0] = cov_ref[0]
    ev = jnp.exp(emb_ref[0, :, pl.ds(i * _ROWS, _ROWS)]) + jnp.exp(ns_ref[0, 0])
    row = jax.lax.broadcasted_iota(jnp.int32, (_ROWS, _ROWS), 0)
    col = jax.lax.broadcasted_iota(jnp.int32, (_ROWS, _ROWS), 1)
    sub = out_ref[0, :, pl.ds(i * _ROWS, _ROWS)]
    out_ref[0, :, pl.ds(i * _ROWS, _ROWS)] = sub + jnp.where(row == col, ev, 0.0)


def kernel(cov, embeddings, noise_scale):
    emb = embeddings[:, :, -1].reshape(_B, 1, _N)
    ns = noise_scale.reshape(1, 1)
    return pl.pallas_call(
        _stripe_kernel,
        grid=(_B, _N // _ROWS),
        in_specs=[
            pl.BlockSpec((1, 1, _N), lambda b, i: (b, 0, 0)),
            pl.BlockSpec((1, 1), lambda b, i: (0, 0)),
            pl.BlockSpec((1, _ROWS, _N), lambda b, i: (b, i, 0)),
        ],
        out_specs=pl.BlockSpec((1, _ROWS, _N), lambda b, i: (b, i, 0)),
        out_shape=jax.ShapeDtypeStruct((_B, _N, _N), jnp.float32),
    )(emb, ns, cov)
